# Initial kernel scaffold; baseline (speedup 1.0000x reference)
import jax
import jax.numpy as jnp
from jax.experimental import pallas as pl


def kernel(node_features, edge_index, W1, b1, g1, be1, W2, b2, g2, be2, Wf, bf):
    raise NotImplementedError("write your pallas kernel here")


# speedup vs baseline: 2.5200x; 2.5200x over previous
import functools

import jax
import jax.numpy as jnp
from jax import lax
from jax.experimental import pallas as pl
from jax.experimental.pallas import tpu as pltpu
from jax.experimental.pallas import tpu_sc as plsc

_DH = 128
_TILES = 16
_CHUNK = 80


def _make_sc_scatter(NP, E):
  rows_per_tile = NP // _TILES
  edges_per_tile = E // _TILES
  nchunk = edges_per_tile // _CHUNK
  mesh = plsc.VectorSubcoreMesh(core_axis_name="c", subcore_axis_name="s")

  @functools.partial(
      pl.kernel,
      mesh=mesh,
      out_type=jax.ShapeDtypeStruct((2 * NP, _DH), jnp.float32),
      scratch_types=[
          pltpu.VMEM((_CHUNK,), jnp.int32),
          pltpu.VMEM((_CHUNK,), jnp.int32),
          pltpu.VMEM((_CHUNK, _DH), jnp.float32),
          pltpu.VMEM_SHARED((NP, _DH), jnp.float32),
          pltpu.SemaphoreType.DMA,
      ],
  )
  def sc_scatter(u_hbm, src_hbm, dst_hbm, z_hbm, out_hbm,
                 src_v, dst_v, rows_v, acc_sh, sem):
    c = lax.axis_index("c")
    s = lax.axis_index("s")
    row0 = s * rows_per_tile
    pltpu.sync_copy(z_hbm, acc_sh.at[pl.ds(row0, rows_per_tile)])
    plsc.subcore_barrier()

    ebase = s * edges_per_tile

    def step(i, carry):
      off = ebase + i * _CHUNK
      pltpu.sync_copy(src_hbm.at[pl.ds(c * E + off, _CHUNK)], src_v)
      pltpu.sync_copy(dst_hbm.at[pl.ds(off, _CHUNK)], dst_v)
      pltpu.async_copy(u_hbm.at[src_v], rows_v, sem).wait()
      pltpu.sync_copy(rows_v, acc_sh.at[dst_v], add=True)
      return carry

    lax.fori_loop(0, nchunk, step, 0)
    plsc.subcore_barrier()
    pltpu.sync_copy(acc_sh.at[pl.ds(row0, rows_per_tile)],
                    out_hbm.at[pl.ds(c * NP + row0, rows_per_tile)])

  return sc_scatter


def _t0_body(x_ref, deg_ref, u0_ref, nrm_ref):
  nrm = lax.rsqrt(jnp.maximum(deg_ref[...], 1.0))
  nrm_ref[...] = nrm
  u0_ref[...] = x_ref[...] * nrm[:, 0:1]


def _tscale_body(s_ref, n_ref, f_ref, u_ref):
  a = n_ref[:, 0:1]
  f = s_ref[...] * a
  f_ref[...] = f
  u_ref[...] = f * a


def _tmm_body(f_ref, w_ref, b_ref, y_ref, ps_ref, pq_ref, acc_ref, *, nk):
  k = pl.program_id(1)

  @pl.when(k == 0)
  def _():
    acc_ref[...] = jnp.zeros_like(acc_ref)

  acc_ref[...] += jnp.dot(f_ref[0], w_ref[0],
                          preferred_element_type=jnp.float32)

  @pl.when(k == nk - 1)
  def _():
    y = acc_ref[...] + b_ref[...]
    y_ref[...] = y
    ps_ref[...] = jnp.sum(y, axis=0, keepdims=True)[None]
    pq_ref[...] = jnp.sum(y * y, axis=0, keepdims=True)[None]


def _bn_scale_shift(ps_ref, pq_ref, g_ref, be_ref, n):
  mu = jnp.sum(ps_ref[...], axis=0) / n
  ex2 = jnp.sum(pq_ref[...], axis=0) / n
  var = ex2 - mu * mu
  scale = g_ref[...] * lax.rsqrt(var + 1e-5)
  shift = be_ref[...] - mu * scale
  return scale, shift


def _tbn_body(y_ref, ps_ref, pq_ref, g_ref, be_ref, n_ref, h_ref, u_ref, *, n):
  scale, shift = _bn_scale_shift(ps_ref, pq_ref, g_ref, be_ref, n)
  h = jax.nn.relu(y_ref[...] * scale + shift)
  h_ref[...] = h
  u_ref[...] = h * n_ref[:, 0:1]


def _tfinal_body(y_ref, ps_ref, pq_ref, g_ref, be_ref, w_ref, bf_ref, x_ref,
                 o_ref, *, n):
  scale, shift = _bn_scale_shift(ps_ref, pq_ref, g_ref, be_ref, n)
  z = jax.nn.relu(y_ref[...] * scale + shift)
  o_ref[...] = (jnp.dot(z, w_ref[...], preferred_element_type=jnp.float32)
                + bf_ref[...] + x_ref[...])


def kernel(node_features, edge_index, W1, b1, g1, be1, W2, b2, g2, be2,
           Wf, bf):
  x = node_features
  N, D = x.shape
  E = edge_index.shape[1]
  K = W1.shape[1] // D - 1
  NK = 2 * (K + 1)
  TN = 1000
  GN = N // TN
  f32 = jnp.float32

  NP = ((N + 127) // 128) * 128
  TS = 2 * NP // 32
  zpad = jnp.zeros((NP - N, _DH), f32)

  src = edge_index[0]
  dst = edge_index[1]
  src2 = jnp.concatenate([src, src + NP])
  zeros_tile = jnp.zeros((NP // _TILES, _DH), f32)

  sc_scatter = _make_sc_scatter(NP, E)

  def split(h):
    return jnp.concatenate([h[:, :_DH], zpad, h[:, _DH:], zpad], axis=0)

  degrep = sc_scatter(jnp.ones((2 * NP, _DH), f32), src2, dst, zeros_tile)

  u0, nrmrep = pl.pallas_call(
      _t0_body,
      grid=(GN,),
      in_specs=[
          pl.BlockSpec((TN, D), lambda n: (n, 0)),
          pl.BlockSpec((TN, _DH), lambda n: (n, 0)),
      ],
      out_specs=[
          pl.BlockSpec((TN, D), lambda n: (n, 0)),
          pl.BlockSpec((TN, _DH), lambda n: (n, 0)),
      ],
      out_shape=[
          jax.ShapeDtypeStruct((N, D), f32),
          jax.ShapeDtypeStruct((N, _DH), f32),
      ],
  )(x, degrep[:N])
  nrm2N = jnp.concatenate([nrmrep, zpad, nrmrep, zpad], axis=0)

  tscale = pl.pallas_call(
      _tscale_body,
      grid=(2 * NP // TS,),
      in_specs=[pl.BlockSpec((TS, _DH), lambda n: (n, 0))] * 2,
      out_specs=[pl.BlockSpec((TS, _DH), lambda n: (n, 0))] * 2,
      out_shape=[jax.ShapeDtypeStruct((2 * NP, _DH), f32)] * 2,
  )

  def tag_stack(h0, u0s):
    parts = [split(h0)]
    u = u0s
    for _ in range(K):
      s2 = sc_scatter(u, src2, dst, zeros_tile)
      f, u = tscale(s2, nrm2N)
      parts.append(f)
    stk = jnp.stack(parts).reshape(K + 1, 2, NP, _DH)
    return stk[:, :, :N, :].reshape(NK, N, _DH)

  tmm = pl.pallas_call(
      functools.partial(_tmm_body, nk=NK),
      grid=(GN, NK),
      in_specs=[
          pl.BlockSpec((1, TN, _DH), lambda n, k: (k, n, 0)),
          pl.BlockSpec((1, _DH, D), lambda n, k: (k, 0, 0)),
          pl.BlockSpec((1, D), lambda n, k: (0, 0)),
      ],
      out_specs=[
          pl.BlockSpec((TN, D), lambda n, k: (n, 0)),
          pl.BlockSpec((1, 1, D), lambda n, k: (n, 0, 0)),
          pl.BlockSpec((1, 1, D), lambda n, k: (n, 0, 0)),
      ],
      out_shape=[
          jax.ShapeDtypeStruct((N, D), f32),
          jax.ShapeDtypeStruct((GN, 1, D), f32),
          jax.ShapeDtypeStruct((GN, 1, D), f32),
      ],
      scratch_shapes=[pltpu.VMEM((TN, D), f32)],
  )

  tbn = pl.pallas_call(
      functools.partial(_tbn_body, n=float(N)),
      grid=(GN,),
      in_specs=[
          pl.BlockSpec((TN, D), lambda n: (n, 0)),
          pl.BlockSpec((GN, 1, D), lambda n: (0, 0, 0)),
          pl.BlockSpec((GN, 1, D), lambda n: (0, 0, 0)),
          pl.BlockSpec((1, D), lambda n: (0, 0)),
          pl.BlockSpec((1, D), lambda n: (0, 0)),
          pl.BlockSpec((TN, _DH), lambda n: (n, 0)),
      ],
      out_specs=[pl.BlockSpec((TN, D), lambda n: (n, 0))] * 2,
      out_shape=[jax.ShapeDtypeStruct((N, D), f32)] * 2,
  )

  tfinal = pl.pallas_call(
      functools.partial(_tfinal_body, n=float(N)),
      grid=(GN,),
      in_specs=[
          pl.BlockSpec((TN, D), lambda n: (n, 0)),
          pl.BlockSpec((GN, 1, D), lambda n: (0, 0, 0)),
          pl.BlockSpec((GN, 1, D), lambda n: (0, 0, 0)),
          pl.BlockSpec((1, D), lambda n: (0, 0)),
          pl.BlockSpec((1, D), lambda n: (0, 0)),
          pl.BlockSpec((D, D), lambda n: (0, 0)),
          pl.BlockSpec((1, D), lambda n: (0, 0)),
          pl.BlockSpec((TN, D), lambda n: (n, 0)),
      ],
      out_specs=pl.BlockSpec((TN, D), lambda n: (n, 0)),
      out_shape=jax.ShapeDtypeStruct((N, D), f32),
  )

  W1r = W1.T.reshape(NK, _DH, D)
  W2r = W2.T.reshape(NK, _DH, D)

  Fs1 = tag_stack(x, split(u0))
  Y1, ps1, pq1 = tmm(Fs1, W1r, b1.reshape(1, D))
  h1, u1 = tbn(Y1, ps1, pq1, g1.reshape(1, D), be1.reshape(1, D), nrmrep)
  Fs2 = tag_stack(h1, split(u1))
  Y2, ps2, pq2 = tmm(Fs2, W2r, b2.reshape(1, D))
  return tfinal(Y2, ps2, pq2, g2.reshape(1, D), be2.reshape(1, D),
                Wf.T, bf.reshape(1, D), x)
